# Initial kernel scaffold; baseline (speedup 1.0000x reference)
import jax
import jax.numpy as jnp
from jax.experimental import pallas as pl


def kernel(rv, ri, rf):
    raise NotImplementedError("write your pallas kernel here")


# speedup vs baseline: 6.7930x; 6.7930x over previous
import jax
import jax.numpy as jnp
from jax.experimental import pallas as pl

_B, _C, _H, _W = 16, 1, 512, 512
_TOPK = 180
_TOL = 3
_IDX_BITS = 18


def _monotone_key(x):
    b = jax.lax.bitcast_convert_type(x, jnp.int32)
    return jnp.where(b < 0, b ^ jnp.int32(0x7FFFFFFF), b)


def _topk_mask(keys, idx, k):
    minint = jnp.int32(-2147483648)
    one = jnp.int32(1)

    def body_ts(i, ub):
        bit = 31 - i
        ubc = ub | jnp.left_shift(one, bit)
        tsc = ubc ^ minint
        cnt = jnp.sum((keys >= tsc).astype(jnp.int32))
        return jnp.where(cnt >= k, ubc, ub)

    ub = jax.lax.fori_loop(0, 32, body_ts, jnp.int32(0))
    ts = ub ^ minint
    cnt_gt = jnp.sum((keys > ts).astype(jnp.int32))
    r = k - cnt_gt
    tie = keys == ts

    def body_cut(i, ub2):
        bit = (_IDX_BITS - 1) - i
        sc = ub2 | jnp.left_shift(one, bit)
        cnt = jnp.sum((tie & (idx < sc)).astype(jnp.int32))
        return jnp.where(cnt < r, sc, ub2)

    cut = jax.lax.fori_loop(0, _IDX_BITS, body_cut, jnp.int32(0))
    sel = (keys > ts) | (tie & (idx <= cut))
    return sel.astype(jnp.float32)


def _dilate(m):
    h, w = m.shape
    f = m
    for d in (1, 2, 3):
        up = jnp.concatenate([m[d:, :], jnp.zeros((d, w), jnp.float32)], axis=0)
        dn = jnp.concatenate([jnp.zeros((d, w), jnp.float32), m[: h - d, :]], axis=0)
        f = jnp.maximum(f, jnp.maximum(up, dn))
    g = f
    for d in (1, 2, 3):
        lf = jnp.concatenate([f[:, d:], jnp.zeros((h, d), jnp.float32)], axis=1)
        rt = jnp.concatenate([jnp.zeros((h, d), jnp.float32), f[:, : w - d]], axis=1)
        g = jnp.maximum(g, jnp.maximum(lf, rt))
    return g


def _body(rv_ref, ri_ref, rf_ref, out_ref):
    pid = pl.program_id(0)
    xv = rv_ref[0]
    xi = ri_ref[0]
    xf = rf_ref[0]
    h, w = xv.shape
    row = jax.lax.broadcasted_iota(jnp.int32, (h, w), 0)
    col = jax.lax.broadcasted_iota(jnp.int32, (h, w), 1)
    idx = row * w + col

    rv_m = _topk_mask(_monotone_key(xv), idx, _TOPK)
    ri_m = _topk_mask(_monotone_key(xi), idx, _TOPK)
    rf_m = _topk_mask(_monotone_key(xf), idx, 2 * _TOPK)

    src = jnp.maximum(rv_m, ri_m)
    dil = _dilate(rf_m)
    cover = jnp.sum(src * dil)
    denom = jnp.maximum(jnp.sum(src), 1.0)
    contrib = (1.0 - cover / denom) * jnp.float32(1.0 / _B)

    @pl.when(pid == 0)
    def _():
        out_ref[...] = jnp.zeros_like(out_ref)

    out_ref[...] += contrib


def kernel(rv, ri, rf):
    b, c, h, w = rv.shape
    rv3 = rv.reshape(b, c * h, w)
    ri3 = ri.reshape(b, c * h, w)
    rf3 = rf.reshape(b, c * h, w)
    spec = pl.BlockSpec((1, c * h, w), lambda i: (i, 0, 0))
    out = pl.pallas_call(
        _body,
        grid=(b,),
        in_specs=[spec, spec, spec],
        out_specs=pl.BlockSpec((1, 128), lambda i: (0, 0)),
        out_shape=jax.ShapeDtypeStruct((1, 128), jnp.float32),
    )(rv3, ri3, rf3)
    return out[0, 0]


# speedup vs baseline: 11.1544x; 1.6421x over previous
import jax
import jax.numpy as jnp
from jax.experimental import pallas as pl

_B, _C, _H, _W = 16, 1, 512, 512
_TOPK = 180
_TOL = 3
_IDX_BITS = 18
_NJ = 4
_S = 384
_CID_BITS = 11
_MININT = -2147483648


def _monotone_key(x):
    b = jax.lax.bitcast_convert_type(x, jnp.int32)
    return jnp.where(b < 0, b ^ jnp.int32(0x7FFFFFFF), b)


def _kth3(arrs, ks):
    minint = jnp.int32(_MININT)
    one = jnp.int32(1)

    def body(i, ubs):
        bit = 31 - i
        out = []
        for u, a, k in zip(ubs, arrs, ks):
            p = u | jnp.left_shift(one, bit)
            cnt = jnp.sum((a >= (p ^ minint)).astype(jnp.int32))
            out.append(jnp.where(cnt >= k, p, u))
        return tuple(out)

    ubs = jax.lax.fori_loop(0, 32, body, (jnp.int32(0),) * 3)
    return tuple(u ^ minint for u in ubs)


def _cut3(ties, ids, rs, nbits):
    one = jnp.int32(1)

    def body(i, ubs):
        bit = (nbits - 1) - i
        out = []
        for u, tie, idv, r in zip(ubs, ties, ids, rs):
            s = u | jnp.left_shift(one, bit)
            cnt = jnp.sum((tie & (idv < s)).astype(jnp.int32))
            out.append(jnp.where(cnt < r, s, u))
        return tuple(out)

    ubs = jax.lax.fori_loop(0, nbits, body, (jnp.int32(0),) * 3)
    return ubs


def _chunk_maxes(keys):
    cms = [
        jnp.max(keys[:, j * 128 : (j + 1) * 128], axis=1, keepdims=True)
        for j in range(_NJ)
    ]
    return jnp.concatenate(cms, axis=1)


def _compact(keys, selc, pos):
    minint = jnp.int32(_MININT)
    h, w = keys.shape
    ubk = keys ^ minint
    lane_r = jax.lax.broadcasted_iota(jnp.int32, (h, _S), 1)
    hcol = jax.lax.broadcasted_iota(jnp.int32, (h, 1), 0)
    h_lo = (hcol & 255).astype(jnp.float32)
    h_hi = jax.lax.shift_right_logical(hcol, 8).astype(jnp.float32)
    ones_col = jnp.ones((h, 1), jnp.float32)

    gp = [jnp.zeros((_S, 128), jnp.float32) for _ in range(4)]
    ghl = jnp.zeros((_S, 1), jnp.float32)
    ghh = jnp.zeros((_S, 1), jnp.float32)
    gj = jnp.zeros((_S, 1), jnp.float32)
    dn = (((0,), (0,)), ((), ()))
    for j in range(_NJ):
        posj = pos[:, j : j + 1].astype(jnp.int32)
        q = ((lane_r == posj) & (selc[:, j : j + 1] > 0.5)).astype(
            jnp.float32
        )
        ub_j = ubk[:, j * 128 : (j + 1) * 128]
        for p in range(4):
            plane = (
                jax.lax.shift_right_logical(ub_j, 8 * (3 - p)) & 255
            ).astype(jnp.float32)
            gp[p] = gp[p] + jax.lax.dot_general(q, plane, dn)
        ghl = ghl + jax.lax.dot_general(q, h_lo, dn)
        ghh = ghh + jax.lax.dot_general(q, h_hi, dn)
        gj = gj + jax.lax.dot_general(q, ones_col, dn) * float(j)

    ip = [g.astype(jnp.int32) for g in gp]
    cand_ub = (
        jnp.left_shift(ip[0], 24)
        | jnp.left_shift(ip[1], 16)
        | jnp.left_shift(ip[2], 8)
        | ip[3]
    )
    cand_keys = cand_ub ^ minint
    h_r = (ghh * 256.0 + ghl).astype(jnp.int32)
    base = h_r * w + gj.astype(jnp.int32) * 128
    cand_idx = base + jax.lax.broadcasted_iota(jnp.int32, (_S, 128), 1)
    return cand_keys, cand_idx


def _positions(selc, tril):
    s = selc
    rowtot = s[:, 0:1] + s[:, 1:2] + s[:, 2:3] + s[:, 3:4]
    cumex = jax.lax.dot_general(
        tril, rowtot, (((1,), (0,)), ((), ()))
    )
    p0 = cumex
    p1 = p0 + s[:, 0:1]
    p2 = p1 + s[:, 1:2]
    p3 = p2 + s[:, 2:3]
    return jnp.concatenate([p0, p1, p2, p3], axis=1)


def _dilate(m):
    h, w = m.shape
    f = m
    for d in (1, 2, 3):
        up = jnp.concatenate([m[d:, :], jnp.zeros((d, w), jnp.float32)], axis=0)
        dnn = jnp.concatenate([jnp.zeros((d, w), jnp.float32), m[: h - d, :]], axis=0)
        f = jnp.maximum(f, jnp.maximum(up, dnn))
    g = f
    for d in (1, 2, 3):
        lf = jnp.concatenate([f[:, d:], jnp.zeros((h, d), jnp.float32)], axis=1)
        rt = jnp.concatenate([jnp.zeros((h, d), jnp.float32), f[:, : w - d]], axis=1)
        g = jnp.maximum(g, jnp.maximum(lf, rt))
    return g


def _body(rv_ref, ri_ref, rf_ref, out_ref):
    pid = pl.program_id(0)
    h, w = rv_ref.shape[1], rv_ref.shape[2]
    row = jax.lax.broadcasted_iota(jnp.int32, (h, w), 0)
    col = jax.lax.broadcasted_iota(jnp.int32, (h, w), 1)
    idx = row * w + col
    tril = (
        jax.lax.broadcasted_iota(jnp.int32, (h, h), 0)
        > jax.lax.broadcasted_iota(jnp.int32, (h, h), 1)
    ).astype(jnp.float32)
    cid = (
        jax.lax.broadcasted_iota(jnp.int32, (h, _NJ), 0) * _NJ
        + jax.lax.broadcasted_iota(jnp.int32, (h, _NJ), 1)
    )

    keys = [_monotone_key(r[0]) for r in (rv_ref, ri_ref, rf_ref)]
    ckeys = [_chunk_maxes(kk) for kk in keys]

    cts = _kth3(ckeys, (_S, _S, _S))
    ctie = [ck == t for ck, t in zip(ckeys, cts)]
    crs = [
        jnp.int32(_S) - jnp.sum((ck > t).astype(jnp.int32))
        for ck, t in zip(ckeys, cts)
    ]
    ccuts = _cut3(ctie, (cid, cid, cid), crs, _CID_BITS)
    selcs = [
        ((ck > t) | (ti & (cid <= cu))).astype(jnp.float32)
        for ck, t, ti, cu in zip(ckeys, cts, ctie, ccuts)
    ]

    cands = [
        _compact(kk, sc, _positions(sc, tril)) for kk, sc in zip(keys, selcs)
    ]
    ckq = [ckv for ckv, _ in cands]
    cix = [civ for _, civ in cands]
    kks = (_TOPK, _TOPK, 2 * _TOPK)
    tss = _kth3(ckq, kks)
    ties = [cq == t for cq, t in zip(ckq, tss)]
    rrs = [
        jnp.int32(k) - jnp.sum((cq > t).astype(jnp.int32))
        for cq, t, k in zip(ckq, tss, kks)
    ]
    cuts = _cut3(ties, cix, rrs, _IDX_BITS)

    masks = [
        ((kk > t) | ((kk == t) & (idx <= cu))).astype(jnp.float32)
        for kk, t, cu in zip(keys, tss, cuts)
    ]

    src = jnp.maximum(masks[0], masks[1])
    dil = _dilate(masks[2])
    cover = jnp.sum(src * dil)
    denom = jnp.maximum(jnp.sum(src), 1.0)
    contrib = (1.0 - cover / denom) * jnp.float32(1.0 / _B)

    @pl.when(pid == 0)
    def _():
        out_ref[...] = jnp.zeros_like(out_ref)

    out_ref[...] += contrib


def kernel(rv, ri, rf):
    b, c, h, w = rv.shape
    rv3 = rv.reshape(b, c * h, w)
    ri3 = ri.reshape(b, c * h, w)
    rf3 = rf.reshape(b, c * h, w)
    spec = pl.BlockSpec((1, c * h, w), lambda i: (i, 0, 0))
    out = pl.pallas_call(
        _body,
        grid=(b,),
        in_specs=[spec, spec, spec],
        out_specs=pl.BlockSpec((1, 128), lambda i: (0, 0)),
        out_shape=jax.ShapeDtypeStruct((1, 128), jnp.float32),
    )(rv3, ri3, rf3)
    return out[0, 0]
